# Initial kernel scaffold; baseline (speedup 1.0000x reference)
import jax
import jax.numpy as jnp
from jax.experimental import pallas as pl


def kernel(x, emb_a, emb_b, W1, b1, W2, b2, W3, b3, Wo, bo):
    raise NotImplementedError("write your pallas kernel here")


# speedup vs baseline: 1.2775x; 1.2775x over previous
import functools

import jax
import jax.numpy as jnp
from jax import lax
from jax.experimental import pallas as pl
from jax.experimental.pallas import tpu as pltpu
from jax.experimental.pallas import tpu_sc as plsc

_ED = 16
_IDX_CHUNK = 128


@functools.cache
def _sc_gather2(B: int, V: int):
    info = plsc.get_sparse_core_info()
    nc, ns = info.num_cores, info.num_subcores
    nw = nc * ns
    bpw = B // nw
    n_chunks = bpw // _IDX_CHUNK
    assert bpw % _IDX_CHUNK == 0 and B % nw == 0
    mesh = plsc.VectorSubcoreMesh(core_axis_name="c", subcore_axis_name="s")

    @functools.partial(
        pl.kernel,
        out_type=(
            jax.ShapeDtypeStruct((B, _ED), jnp.float32),
            jax.ShapeDtypeStruct((B, _ED), jnp.float32),
        ),
        mesh=mesh,
        compiler_params=pltpu.CompilerParams(use_tc_tiling_on_sc=False),
        scratch_types=[
            pltpu.VMEM((n_chunks, _IDX_CHUNK), jnp.int32),
            pltpu.VMEM((n_chunks, _IDX_CHUNK), jnp.int32),
            pltpu.VMEM((bpw, _ED), jnp.float32),
            pltpu.VMEM((bpw, _ED), jnp.float32),
            pltpu.SemaphoreType.DMA,
            pltpu.SemaphoreType.DMA,
        ],
    )
    def gather2(ids_a_hbm, ids_b_hbm, tab_a, tab_b, out_a, out_b,
                idx_a, idx_b, rows_a, rows_b, sem_a, sem_b):
        wid = lax.axis_index("s") * nc + lax.axis_index("c")
        base = wid * bpw
        for j in range(n_chunks):
            pltpu.sync_copy(ids_a_hbm.at[pl.ds(base + j * _IDX_CHUNK, _IDX_CHUNK)],
                            idx_a.at[j])
            pltpu.sync_copy(ids_b_hbm.at[pl.ds(base + j * _IDX_CHUNK, _IDX_CHUNK)],
                            idx_b.at[j])
        copies = []
        for j in range(n_chunks):
            copies.append(pltpu.async_copy(
                tab_a.at[idx_a.at[j]],
                rows_a.at[pl.ds(j * _IDX_CHUNK, _IDX_CHUNK)], sem_a))
            copies.append(pltpu.async_copy(
                tab_b.at[idx_b.at[j]],
                rows_b.at[pl.ds(j * _IDX_CHUNK, _IDX_CHUNK)], sem_b))
        for c in copies:
            c.wait()
        pltpu.sync_copy(rows_a, out_a.at[pl.ds(base, bpw)])
        pltpu.sync_copy(rows_b, out_b.at[pl.ds(base, bpw)])

    return gather2


def _mlp_body(ea, eb, num, w1a, w1b, w1n, b1, w2, b2, w3, b3, wo, bo, out):
    h = jnp.dot(ea[...], w1a[...], preferred_element_type=jnp.float32)
    h = h + jnp.dot(eb[...], w1b[...], preferred_element_type=jnp.float32)
    h = h + jnp.dot(num[...], w1n[...], preferred_element_type=jnp.float32)
    h = jnp.maximum(h + b1[...], 0.0)
    h = jnp.maximum(
        jnp.dot(h, w2[...], preferred_element_type=jnp.float32) + b2[...], 0.0)
    h = jnp.maximum(
        jnp.dot(h, w3[...], preferred_element_type=jnp.float32) + b3[...], 0.0)
    z = jnp.sum(h * wo[...], axis=1, keepdims=True) + bo[...]
    out[...] = 1.0 / (1.0 + jnp.exp(-z))


@functools.cache
def _mlp_call(B: int, F: int, blk: int):
    full = lambda shape: pl.BlockSpec(shape, lambda i: (0, 0))
    return pl.pallas_call(
        _mlp_body,
        grid=(B // blk,),
        in_specs=[
            pl.BlockSpec((blk, _ED), lambda i: (i, 0)),
            pl.BlockSpec((blk, _ED), lambda i: (i, 0)),
            pl.BlockSpec((blk, F), lambda i: (i, 0)),
            full((_ED, 64)),
            full((_ED, 64)),
            full((F, 64)),
            full((1, 64)),
            full((64, 32)),
            full((1, 32)),
            full((32, 16)),
            full((1, 16)),
            full((1, 16)),
            full((1, 1)),
        ],
        out_specs=pl.BlockSpec((blk, 1), lambda i: (i, 0)),
        out_shape=jax.ShapeDtypeStruct((B, 1), jnp.float32),
    )


def kernel(x, emb_a, emb_b, W1, b1, W2, b2, W3, b3, Wo, bo):
    B, C = x.shape
    F = C - 2
    V = emb_a.shape[0]
    ids_a = x[:, 0].astype(jnp.int32)
    ids_b = x[:, 1].astype(jnp.int32)
    numeric = x[:, 2:]
    ea, eb = _sc_gather2(B, V)(ids_a, ids_b, emb_a, emb_b)
    W1T = W1.T
    out = _mlp_call(B, F, 2048)(
        ea, eb, numeric,
        W1T[:_ED], W1T[_ED:2 * _ED], W1T[2 * _ED:],
        b1.reshape(1, 64), W2.T, b2.reshape(1, 32), W3.T, b3.reshape(1, 16),
        Wo, bo.reshape(1, 1))
    return out


# speedup vs baseline: 1.2812x; 1.0029x over previous
import functools

import jax
import jax.numpy as jnp
from jax import lax
from jax.experimental import pallas as pl
from jax.experimental.pallas import tpu as pltpu
from jax.experimental.pallas import tpu_sc as plsc

_ED = 16
_IDX_CHUNK = 128


@functools.cache
def _sc_gather2(B: int, V: int):
    info = plsc.get_sparse_core_info()
    nc, ns = info.num_cores, info.num_subcores
    nw = nc * ns
    bpw = B // nw
    n_chunks = bpw // _IDX_CHUNK
    assert bpw % _IDX_CHUNK == 0 and B % nw == 0
    mesh = plsc.VectorSubcoreMesh(core_axis_name="c", subcore_axis_name="s")

    @functools.partial(
        pl.kernel,
        out_type=(
            jax.ShapeDtypeStruct((B, _ED), jnp.float32),
            jax.ShapeDtypeStruct((B, _ED), jnp.float32),
        ),
        mesh=mesh,
        compiler_params=pltpu.CompilerParams(use_tc_tiling_on_sc=False),
        scratch_types=[
            pltpu.VMEM((bpw,), jnp.int32),
            pltpu.VMEM((bpw,), jnp.int32),
            pltpu.VMEM((bpw, _ED), jnp.float32),
            pltpu.VMEM((bpw, _ED), jnp.float32),
            pltpu.SemaphoreType.DMA,
            pltpu.SemaphoreType.DMA,
        ],
    )
    def gather2(ids_a_hbm, ids_b_hbm, tab_a, tab_b, out_a, out_b,
                idx_a, idx_b, rows_a, rows_b, sem_a, sem_b):
        wid = lax.axis_index("s") * nc + lax.axis_index("c")
        base = wid * bpw
        ca = pltpu.async_copy(ids_a_hbm.at[pl.ds(base, bpw)], idx_a, sem_a)
        cb = pltpu.async_copy(ids_b_hbm.at[pl.ds(base, bpw)], idx_b, sem_b)
        ca.wait()
        cb.wait()
        copies = []
        for j in range(n_chunks):
            copies.append(pltpu.async_copy(
                tab_a.at[idx_a.at[pl.ds(j * _IDX_CHUNK, _IDX_CHUNK)]],
                rows_a.at[pl.ds(j * _IDX_CHUNK, _IDX_CHUNK)], sem_a))
            copies.append(pltpu.async_copy(
                tab_b.at[idx_b.at[pl.ds(j * _IDX_CHUNK, _IDX_CHUNK)]],
                rows_b.at[pl.ds(j * _IDX_CHUNK, _IDX_CHUNK)], sem_b))
        for c in copies:
            c.wait()
        wa = pltpu.async_copy(rows_a, out_a.at[pl.ds(base, bpw)], sem_a)
        wb = pltpu.async_copy(rows_b, out_b.at[pl.ds(base, bpw)], sem_b)
        wa.wait()
        wb.wait()

    return gather2


def _mlp_body(ea, eb, num, w1a, w1b, w1n, b1, w2, b2, w3, b3, wo, bo, out):
    h = jnp.dot(ea[...], w1a[...], preferred_element_type=jnp.float32)
    h = h + jnp.dot(eb[...], w1b[...], preferred_element_type=jnp.float32)
    h = h + jnp.dot(num[...], w1n[...], preferred_element_type=jnp.float32)
    h = jnp.maximum(h + b1[...], 0.0)
    h = jnp.maximum(
        jnp.dot(h, w2[...], preferred_element_type=jnp.float32) + b2[...], 0.0)
    h = jnp.maximum(
        jnp.dot(h, w3[...], preferred_element_type=jnp.float32) + b3[...], 0.0)
    z = jnp.sum(h * wo[...], axis=1, keepdims=True) + bo[...]
    out[...] = 1.0 / (1.0 + jnp.exp(-z))


@functools.cache
def _mlp_call(B: int, F: int, blk: int):
    full = lambda shape: pl.BlockSpec(shape, lambda i: (0, 0))
    return pl.pallas_call(
        _mlp_body,
        grid=(B // blk,),
        in_specs=[
            pl.BlockSpec((blk, _ED), lambda i: (i, 0)),
            pl.BlockSpec((blk, _ED), lambda i: (i, 0)),
            pl.BlockSpec((blk, F), lambda i: (i, 0)),
            full((_ED, 64)),
            full((_ED, 64)),
            full((F, 64)),
            full((1, 64)),
            full((64, 32)),
            full((1, 32)),
            full((32, 16)),
            full((1, 16)),
            full((1, 16)),
            full((1, 1)),
        ],
        out_specs=pl.BlockSpec((blk, 1), lambda i: (i, 0)),
        out_shape=jax.ShapeDtypeStruct((B, 1), jnp.float32),
    )


def kernel(x, emb_a, emb_b, W1, b1, W2, b2, W3, b3, Wo, bo):
    B, C = x.shape
    F = C - 2
    V = emb_a.shape[0]
    ids_a = x[:, 0].astype(jnp.int32)
    ids_b = x[:, 1].astype(jnp.int32)
    numeric = x[:, 2:]
    ea, eb = _sc_gather2(B, V)(ids_a, ids_b, emb_a, emb_b)
    W1T = W1.T
    out = _mlp_call(B, F, 2048)(
        ea, eb, numeric,
        W1T[:_ED], W1T[_ED:2 * _ED], W1T[2 * _ED:],
        b1.reshape(1, 64), W2.T, b2.reshape(1, 32), W3.T, b3.reshape(1, 16),
        Wo, bo.reshape(1, 1))
    return out
